# baseline (speedup 1.0000x reference)
import jax
import jax.numpy as jnp
from jax.experimental import pallas as pl
from jax.experimental.pallas import tpu as pltpu

_B, _N, _DIN = 16, 1024, 128
_GPB = 2
_NB = _B // _GPB
_H = _N // 2


def _lrelu(x):
    return jnp.where(x >= 0, x, x * 0.01)


def _pass0_body(gt_ref, gb_ref, x_ref, w0_ref, b0_ref, h1_ref):
    for j in range(_GPB):
        t = jnp.dot(x_ref[j], w0_ref[...],
                    preferred_element_type=jnp.float32).astype(jnp.bfloat16)
        for g_ref, sl in ((gt_ref, slice(0, _H)), (gb_ref, slice(_H, _N))):
            a = jnp.dot(g_ref[j, 0], t, preferred_element_type=jnp.float32)
            h1_ref[j, sl] = _lrelu(a + b0_ref[...]).astype(jnp.bfloat16)


def _pass_mid_body(gt_ref, gb_ref, h_ref, w_ref, b_ref, o_ref):
    for j in range(_GPB):
        h = h_ref[j]
        for g_ref, sl in ((gt_ref, slice(0, _H)), (gb_ref, slice(_H, _N))):
            a = jnp.dot(g_ref[j, 0], h, preferred_element_type=jnp.float32)
            z = jnp.dot(a.astype(jnp.bfloat16), w_ref[...],
                        preferred_element_type=jnp.float32) + b_ref[...]
            o_ref[j, sl] = _lrelu(z).astype(jnp.bfloat16)


def _pass3_body(gt_ref, gb_ref, h_ref, w3_ref, b3_ref, wagg_ref, bagg_ref,
                pool_ref):
    for j in range(_GPB):
        h = h_ref[j]
        acc = None
        for g_ref in (gt_ref, gb_ref):
            a = jnp.dot(g_ref[j, 0], h, preferred_element_type=jnp.float32)
            h4 = _lrelu(jnp.dot(a.astype(jnp.bfloat16), w3_ref[...],
                                preferred_element_type=jnp.float32)
                        + b3_ref[...])
            h5 = _lrelu(jnp.dot(h4.astype(jnp.bfloat16), wagg_ref[...],
                                preferred_element_type=jnp.float32)
                        + bagg_ref[...])
            part = jnp.sum(h5, axis=0, keepdims=True)
            acc = part if acc is None else acc + part
        pool_ref[j] = acc


def _head_body(p_ref, wfc_ref, bfc_ref, wout_ref, bout_ref, o_ref):
    z = _lrelu(jnp.dot(p_ref[...], wfc_ref[...],
                       preferred_element_type=jnp.float32) + bfc_ref[...])
    o_ref[...] = (jnp.dot(z, wout_ref[...], preferred_element_type=jnp.float32)
                  + bout_ref[...])


def _full(shape):
    return pl.BlockSpec(shape, lambda b: tuple(0 for _ in shape))


def _step_block(shape):
    return pl.BlockSpec(shape, lambda b: (b,) + tuple(0 for _ in shape[1:]))


def _half_block(half):
    return pl.BlockSpec((_GPB, 1, _H, _N), lambda b: (b, half, 0, 0))


def kernel(adj_list, feats, W_conv0, b_conv0, W_conv1, b_conv1, W_conv2,
           b_conv2, W_conv3, b_conv3, W_agg, b_agg, W_fc, b_fc, W_out, b_out):
    G = jnp.squeeze(adj_list, axis=-1).astype(jnp.bfloat16)
    G4 = jnp.reshape(G, (_B, 2, _H, _N))
    b0 = jnp.reshape(b_conv0, (1, -1))
    b1 = jnp.reshape(b_conv1, (1, -1))
    b2 = jnp.reshape(b_conv2, (1, -1))
    b3 = jnp.reshape(b_conv3, (1, -1))
    bagg = jnp.reshape(b_agg, (1, -1))
    bfc = jnp.reshape(b_fc, (1, -1))
    bout = jnp.reshape(b_out, (1, -1))

    params = pltpu.CompilerParams(dimension_semantics=("arbitrary",))

    def gspecs():
        return [_half_block(0), _half_block(1)]

    h1 = pl.pallas_call(
        _pass0_body,
        grid=(_NB,),
        in_specs=gspecs() + [
            _step_block((_GPB, _N, _DIN)),
            _full(W_conv0.shape),
            _full(b0.shape),
        ],
        out_specs=_step_block((_GPB, _N, 64)),
        out_shape=jax.ShapeDtypeStruct((_B, _N, 64), jnp.bfloat16),
        compiler_params=params,
    )(G4, G4, feats, W_conv0, b0)

    def mid(h, W, b, dout):
        return pl.pallas_call(
            _pass_mid_body,
            grid=(_NB,),
            in_specs=gspecs() + [
                _step_block((_GPB, _N, h.shape[-1])),
                _full(W.shape),
                _full(b.shape),
            ],
            out_specs=_step_block((_GPB, _N, dout)),
            out_shape=jax.ShapeDtypeStruct((_B, _N, dout), jnp.bfloat16),
            compiler_params=params,
        )(G4, G4, h, W, b)

    h2 = mid(h1, W_conv1, b1, 64)
    h3 = mid(h2, W_conv2, b2, 128)

    pool = pl.pallas_call(
        _pass3_body,
        grid=(_NB,),
        in_specs=gspecs() + [
            _step_block((_GPB, _N, 128)),
            _full(W_conv3.shape),
            _full(b3.shape),
            _full(W_agg.shape),
            _full(bagg.shape),
        ],
        out_specs=_step_block((_GPB, 1, 72)),
        out_shape=jax.ShapeDtypeStruct((_B, 1, 72), jnp.float32),
        compiler_params=params,
    )(G4, G4, h3, W_conv3, b3, W_agg, bagg)
    pool = jnp.reshape(pool, (_B, 72))

    out = pl.pallas_call(
        _head_body,
        grid=(1,),
        in_specs=[
            _full((_B, 72)),
            _full(W_fc.shape),
            _full(bfc.shape),
            _full(W_out.shape),
            _full(bout.shape),
        ],
        out_specs=_full((_B, 64)),
        out_shape=jax.ShapeDtypeStruct((_B, 64), jnp.float32),
        compiler_params=params,
    )(pool, W_fc, bfc, W_out, bout)

    return out


# speedup vs baseline: 1.6158x; 1.6158x over previous
import jax
import jax.numpy as jnp
from jax.experimental import pallas as pl
from jax.experimental.pallas import tpu as pltpu

_B, _N, _DIN = 16, 1024, 128
_GPB = 4
_NB = _B // _GPB


def _lrelu(x):
    return jnp.where(x >= 0, x, x * 0.01)


def _pass0_body(g_ref, x_ref, w0_ref, b0_ref, h1_ref):
    for j in range(_GPB):
        t = jnp.dot(x_ref[j], w0_ref[...], preferred_element_type=jnp.float32)
        a = jnp.dot(g_ref[j], t.astype(jnp.bfloat16),
                    preferred_element_type=jnp.float32)
        h1_ref[j] = _lrelu(a + b0_ref[...]).astype(jnp.bfloat16)


def _pass_mid_body(g_ref, h_ref, w_ref, b_ref, o_ref):
    for j in range(_GPB):
        a = jnp.dot(g_ref[j], h_ref[j], preferred_element_type=jnp.float32)
        z = jnp.dot(a.astype(jnp.bfloat16), w_ref[...],
                    preferred_element_type=jnp.float32) + b_ref[...]
        o_ref[j] = _lrelu(z).astype(jnp.bfloat16)


def _pass3_body(g_ref, h_ref, w3_ref, b3_ref, wagg_ref, bagg_ref, pool_ref):
    for j in range(_GPB):
        a = jnp.dot(g_ref[j], h_ref[j], preferred_element_type=jnp.float32)
        h4 = _lrelu(jnp.dot(a.astype(jnp.bfloat16), w3_ref[...],
                            preferred_element_type=jnp.float32) + b3_ref[...])
        h5 = _lrelu(jnp.dot(h4.astype(jnp.bfloat16), wagg_ref[...],
                            preferred_element_type=jnp.float32) + bagg_ref[...])
        pool_ref[j] = jnp.sum(h5, axis=0, keepdims=True)


def _head_body(p_ref, wfc_ref, bfc_ref, wout_ref, bout_ref, o_ref):
    z = _lrelu(jnp.dot(p_ref[...], wfc_ref[...],
                       preferred_element_type=jnp.float32) + bfc_ref[...])
    o_ref[...] = (jnp.dot(z, wout_ref[...], preferred_element_type=jnp.float32)
                  + bout_ref[...])


def _full(shape):
    return pl.BlockSpec(shape, lambda b: tuple(0 for _ in shape))


def _step_block(shape):
    return pl.BlockSpec(shape, lambda b: (b,) + tuple(0 for _ in shape[1:]))


def kernel(adj_list, feats, W_conv0, b_conv0, W_conv1, b_conv1, W_conv2,
           b_conv2, W_conv3, b_conv3, W_agg, b_agg, W_fc, b_fc, W_out, b_out):
    G = jnp.squeeze(adj_list, axis=-1).astype(jnp.bfloat16)
    b0 = jnp.reshape(b_conv0, (1, -1))
    b1 = jnp.reshape(b_conv1, (1, -1))
    b2 = jnp.reshape(b_conv2, (1, -1))
    b3 = jnp.reshape(b_conv3, (1, -1))
    bagg = jnp.reshape(b_agg, (1, -1))
    bfc = jnp.reshape(b_fc, (1, -1))
    bout = jnp.reshape(b_out, (1, -1))

    params = pltpu.CompilerParams(dimension_semantics=("arbitrary",))

    h1 = pl.pallas_call(
        _pass0_body,
        grid=(_NB,),
        in_specs=[
            _step_block((_GPB, _N, _N)),
            _step_block((_GPB, _N, _DIN)),
            _full(W_conv0.shape),
            _full(b0.shape),
        ],
        out_specs=_step_block((_GPB, _N, 64)),
        out_shape=jax.ShapeDtypeStruct((_B, _N, 64), jnp.bfloat16),
        compiler_params=params,
    )(G, feats, W_conv0, b0)

    def mid(h, W, b, dout):
        return pl.pallas_call(
            _pass_mid_body,
            grid=(_NB,),
            in_specs=[
                _step_block((_GPB, _N, _N)),
                _step_block((_GPB, _N, h.shape[-1])),
                _full(W.shape),
                _full(b.shape),
            ],
            out_specs=_step_block((_GPB, _N, dout)),
            out_shape=jax.ShapeDtypeStruct((_B, _N, dout), jnp.bfloat16),
            compiler_params=params,
        )(G, h, W, b)

    h2 = mid(h1, W_conv1, b1, 64)
    h3 = mid(h2, W_conv2, b2, 128)

    pool = pl.pallas_call(
        _pass3_body,
        grid=(_NB,),
        in_specs=[
            _step_block((_GPB, _N, _N)),
            _step_block((_GPB, _N, 128)),
            _full(W_conv3.shape),
            _full(b3.shape),
            _full(W_agg.shape),
            _full(bagg.shape),
        ],
        out_specs=_step_block((_GPB, 1, 72)),
        out_shape=jax.ShapeDtypeStruct((_B, 1, 72), jnp.float32),
        compiler_params=params,
    )(G, h3, W_conv3, b3, W_agg, bagg)
    pool = jnp.reshape(pool, (_B, 72))

    out = pl.pallas_call(
        _head_body,
        grid=(1,),
        in_specs=[
            _full((_B, 72)),
            _full(W_fc.shape),
            _full(bfc.shape),
            _full(W_out.shape),
            _full(bout.shape),
        ],
        out_specs=_full((_B, 64)),
        out_shape=jax.ShapeDtypeStruct((_B, 64), jnp.float32),
        compiler_params=params,
    )(pool, W_fc, bfc, W_out, bout)

    return out


# speedup vs baseline: 1.7704x; 1.0957x over previous
import jax
import jax.numpy as jnp
from jax.experimental import pallas as pl
from jax.experimental.pallas import tpu as pltpu

_B, _N, _DIN = 16, 1024, 128
_GPB = 2
_NB = _B // _GPB


def _lrelu(x):
    return jnp.where(x >= 0, x, x * 0.01)


def _bf(x):
    return x.astype(jnp.bfloat16)


def _main_body(g_ref, x_ref, w0_ref, b0_ref, w1_ref, b1_ref, w2_ref, b2_ref,
               w3_ref, b3_ref, wagg_ref, bagg_ref, pool_ref):
    for j in range(_GPB):
        g = g_ref[j]
        t = _bf(jnp.dot(x_ref[j], w0_ref[...],
                        preferred_element_type=jnp.float32))
        a = jnp.dot(g, t, preferred_element_type=jnp.float32)
        h = _bf(_lrelu(a + b0_ref[...]))
        for w_ref, b_ref in ((w1_ref, b1_ref), (w2_ref, b2_ref)):
            a = jnp.dot(g, h, preferred_element_type=jnp.float32)
            z = jnp.dot(_bf(a), w_ref[...],
                        preferred_element_type=jnp.float32) + b_ref[...]
            h = _bf(_lrelu(z))
        a = jnp.dot(g, h, preferred_element_type=jnp.float32)
        h4 = _lrelu(jnp.dot(_bf(a), w3_ref[...],
                            preferred_element_type=jnp.float32) + b3_ref[...])
        h5 = _lrelu(jnp.dot(_bf(h4), wagg_ref[...],
                            preferred_element_type=jnp.float32)
                    + bagg_ref[...])
        pool_ref[j] = jnp.sum(h5, axis=0, keepdims=True)


def _head_body(p_ref, wfc_ref, bfc_ref, wout_ref, bout_ref, o_ref):
    z = _lrelu(jnp.dot(p_ref[...], wfc_ref[...],
                       preferred_element_type=jnp.float32) + bfc_ref[...])
    o_ref[...] = (jnp.dot(z, wout_ref[...], preferred_element_type=jnp.float32)
                  + bout_ref[...])


def _full(shape):
    return pl.BlockSpec(shape, lambda b: tuple(0 for _ in shape))


def _step_block(shape):
    return pl.BlockSpec(shape, lambda b: (b,) + tuple(0 for _ in shape[1:]))


def kernel(adj_list, feats, W_conv0, b_conv0, W_conv1, b_conv1, W_conv2,
           b_conv2, W_conv3, b_conv3, W_agg, b_agg, W_fc, b_fc, W_out, b_out):
    G = jnp.squeeze(adj_list, axis=-1).astype(jnp.bfloat16)
    b0 = jnp.reshape(b_conv0, (1, -1))
    b1 = jnp.reshape(b_conv1, (1, -1))
    b2 = jnp.reshape(b_conv2, (1, -1))
    b3 = jnp.reshape(b_conv3, (1, -1))
    bagg = jnp.reshape(b_agg, (1, -1))
    bfc = jnp.reshape(b_fc, (1, -1))
    bout = jnp.reshape(b_out, (1, -1))

    params = pltpu.CompilerParams(dimension_semantics=("arbitrary",))

    pool = pl.pallas_call(
        _main_body,
        grid=(_NB,),
        in_specs=[
            _step_block((_GPB, _N, _N)),
            _step_block((_GPB, _N, _DIN)),
            _full(W_conv0.shape), _full(b0.shape),
            _full(W_conv1.shape), _full(b1.shape),
            _full(W_conv2.shape), _full(b2.shape),
            _full(W_conv3.shape), _full(b3.shape),
            _full(W_agg.shape), _full(bagg.shape),
        ],
        out_specs=_step_block((_GPB, 1, 72)),
        out_shape=jax.ShapeDtypeStruct((_B, 1, 72), jnp.float32),
        compiler_params=params,
    )(G, feats, W_conv0, b0, W_conv1, b1, W_conv2, b2, W_conv3, b3,
      W_agg, bagg)
    pool = jnp.reshape(pool, (_B, 72))

    out = pl.pallas_call(
        _head_body,
        grid=(1,),
        in_specs=[
            _full((_B, 72)),
            _full(W_fc.shape),
            _full(bfc.shape),
            _full(W_out.shape),
            _full(bout.shape),
        ],
        out_specs=_full((_B, 64)),
        out_shape=jax.ShapeDtypeStruct((_B, 64), jnp.float32),
        compiler_params=params,
    )(pool, W_fc, bfc, W_out, bout)

    return out


# speedup vs baseline: 1.8749x; 1.0590x over previous
import jax
import jax.numpy as jnp
from jax.experimental import pallas as pl
from jax.experimental.pallas import tpu as pltpu

_B, _N, _DIN = 16, 1024, 128
_GPB = 2
_NB = _B // _GPB


def _lrelu(x):
    return jnp.where(x >= 0, x, x * 0.01)


def _bf(x):
    return x.astype(jnp.bfloat16)


def _main_body(g_ref, x_ref, w0_ref, b0_ref, w1_ref, b1_ref, w2_ref, b2_ref,
               w3_ref, b3_ref, wagg_ref, bagg_ref, pool_ref):
    hs = []
    for j in range(_GPB):
        t = _bf(jnp.dot(x_ref[j], w0_ref[...],
                        preferred_element_type=jnp.float32))
        a = jnp.dot(g_ref[j], t, preferred_element_type=jnp.float32)
        hs.append(_bf(_lrelu(a + b0_ref[...])))
    for w_ref, b_ref in ((w1_ref, b1_ref), (w2_ref, b2_ref)):
        nxt = []
        for j in range(_GPB):
            a = jnp.dot(g_ref[j], hs[j], preferred_element_type=jnp.float32)
            z = jnp.dot(_bf(a), w_ref[...],
                        preferred_element_type=jnp.float32) + b_ref[...]
            nxt.append(_bf(_lrelu(z)))
        hs = nxt
    for j in range(_GPB):
        a = jnp.dot(g_ref[j], hs[j], preferred_element_type=jnp.float32)
        h4 = _lrelu(jnp.dot(_bf(a), w3_ref[...],
                            preferred_element_type=jnp.float32) + b3_ref[...])
        h5 = _lrelu(jnp.dot(_bf(h4), wagg_ref[...],
                            preferred_element_type=jnp.float32)
                    + bagg_ref[...])
        pool_ref[j] = jnp.sum(h5, axis=0, keepdims=True)


def _head_body(p_ref, wfc_ref, bfc_ref, wout_ref, bout_ref, o_ref):
    z = _lrelu(jnp.dot(p_ref[...], wfc_ref[...],
                       preferred_element_type=jnp.float32) + bfc_ref[...])
    o_ref[...] = (jnp.dot(z, wout_ref[...], preferred_element_type=jnp.float32)
                  + bout_ref[...])


def _full(shape):
    return pl.BlockSpec(shape, lambda b: tuple(0 for _ in shape))


def _step_block(shape):
    return pl.BlockSpec(shape, lambda b: (b,) + tuple(0 for _ in shape[1:]))


def kernel(adj_list, feats, W_conv0, b_conv0, W_conv1, b_conv1, W_conv2,
           b_conv2, W_conv3, b_conv3, W_agg, b_agg, W_fc, b_fc, W_out, b_out):
    G = jnp.squeeze(adj_list, axis=-1).astype(jnp.bfloat16)
    b0 = jnp.reshape(b_conv0, (1, -1))
    b1 = jnp.reshape(b_conv1, (1, -1))
    b2 = jnp.reshape(b_conv2, (1, -1))
    b3 = jnp.reshape(b_conv3, (1, -1))
    bagg = jnp.reshape(b_agg, (1, -1))
    bfc = jnp.reshape(b_fc, (1, -1))
    bout = jnp.reshape(b_out, (1, -1))

    params = pltpu.CompilerParams(dimension_semantics=("arbitrary",))

    pool = pl.pallas_call(
        _main_body,
        grid=(_NB,),
        in_specs=[
            _step_block((_GPB, _N, _N)),
            _step_block((_GPB, _N, _DIN)),
            _full(W_conv0.shape), _full(b0.shape),
            _full(W_conv1.shape), _full(b1.shape),
            _full(W_conv2.shape), _full(b2.shape),
            _full(W_conv3.shape), _full(b3.shape),
            _full(W_agg.shape), _full(bagg.shape),
        ],
        out_specs=_step_block((_GPB, 1, 72)),
        out_shape=jax.ShapeDtypeStruct((_B, 1, 72), jnp.float32),
        compiler_params=params,
    )(G, feats, W_conv0, b0, W_conv1, b1, W_conv2, b2, W_conv3, b3,
      W_agg, bagg)
    pool = jnp.reshape(pool, (_B, 72))

    out = pl.pallas_call(
        _head_body,
        grid=(1,),
        in_specs=[
            _full((_B, 72)),
            _full(W_fc.shape),
            _full(bfc.shape),
            _full(W_out.shape),
            _full(bout.shape),
        ],
        out_specs=_full((_B, 64)),
        out_shape=jax.ShapeDtypeStruct((_B, 64), jnp.float32),
        compiler_params=params,
    )(pool, W_fc, bfc, W_out, bout)

    return out
